# baseline (speedup 1.0000x reference)
import functools

import jax
import jax.numpy as jnp
from jax import lax
from jax.experimental import pallas as pl
from jax.experimental.pallas import tpu as pltpu
from jax.experimental.pallas import tpu_sc as plsc

B, S, D = 4, 4096, 768
LANES = 16
VPR = D // LANES
NW = 32
SEQ_PER_W = S // NW
CS = 16
NCHUNK = SEQ_PER_W // CS
NBUF = 2
AHEAD = 1

_mesh = plsc.VectorSubcoreMesh(core_axis_name="c", subcore_axis_name="s")

_scratch = (
    [pltpu.VMEM((NCHUNK * B * CS,), jnp.int32)]
    + [pltpu.VMEM((B * CS, D), jnp.float32) for _ in range(NBUF)]
    + [pltpu.VMEM((CS, D), jnp.float32) for _ in range(NBUF)]
    + [pltpu.SemaphoreType.DMA for _ in range(3 * NBUF)]
)


@functools.partial(
    pl.kernel,
    mesh=_mesh,
    out_type=jax.ShapeDtypeStruct((B, S, D), jnp.float32),
    scratch_types=_scratch,
)
def _embed(x_hbm, table_hbm, pos_hbm, out_hbm, idx_v, *bufs):
    rows_bufs = bufs[:NBUF]
    pos_bufs = bufs[NBUF:2 * NBUF]
    gsem = bufs[2 * NBUF:3 * NBUF]
    psem = bufs[3 * NBUF:4 * NBUF]
    ssem = bufs[4 * NBUF:5 * NBUF]

    cid = lax.axis_index("c")
    sid = lax.axis_index("s")
    wid = sid * 2 + cid
    wseq = wid * SEQ_PER_W

    def stage(g):
        for b in range(B):
            pltpu.async_copy(
                x_hbm.at[b, pl.ds(wseq + g * CS, CS)],
                idx_v.at[pl.ds(g * B * CS + b * CS, CS)],
                gsem[0],
            )

    def stage_drain(n):
        @pl.loop(0, n * B)
        def _(t):
            pltpu.make_async_copy(
                x_hbm.at[0, pl.ds(0, CS)], idx_v.at[pl.ds(0, CS)], gsem[0]
            ).wait()

    def fire(g, j):
        pltpu.async_copy(
            pos_hbm.at[pl.ds(wseq + g * CS, CS)], pos_bufs[j], psem[j]
        )
        pltpu.async_copy(
            table_hbm.at[idx_v.at[pl.ds(g * B * CS, B * CS)]],
            rows_bufs[j],
            gsem[j],
        )

    for g in range(AHEAD):
        stage(g)
    stage_drain(AHEAD)
    for g in range(AHEAD):
        fire(g, g % NBUF)

    @pl.loop(AHEAD, NCHUNK)
    def _stage_rest(g):
        stage(g)

    stage_drain(NCHUNK - AHEAD)

    @pl.loop(0, NCHUNK, step=NBUF)
    def _main(gout):
        for i in range(NBUF):
            g = gout + i
            pltpu.make_async_copy(
                table_hbm.at[pl.ds(0, B * CS)], rows_bufs[i], gsem[i]
            ).wait()
            pltpu.make_async_copy(
                pos_hbm.at[pl.ds(0, CS)], pos_bufs[i], psem[i]
            ).wait()

            j = (i + AHEAD) % NBUF

            @pl.when(g + AHEAD < NCHUNK)
            def _fire_ahead(g=g, j=j):
                @pl.when(g + AHEAD >= NBUF)
                def _drain_store():
                    for b in range(B):
                        pltpu.make_async_copy(
                            rows_bufs[j].at[pl.ds(b * CS, CS)],
                            out_hbm.at[b, pl.ds(0, CS)],
                            ssem[j],
                        ).wait()

                fire(g + AHEAD, j)

            rows = rows_bufs[i]
            pos = pos_bufs[i]

            def add_row(jj, c2, rows=rows, pos=pos):
                def add_vec(k, c3):
                    off = k * LANES
                    p = pos[jj, pl.ds(off, LANES)]
                    for b in range(B):
                        plsc.addupdate(rows.at[b * CS + jj, pl.ds(off, LANES)], p)
                    return c3

                return lax.fori_loop(0, VPR, add_vec, c2)

            lax.fori_loop(0, CS, add_row, 0)

            for b in range(B):
                pltpu.async_copy(
                    rows.at[pl.ds(b * CS, CS)],
                    out_hbm.at[b, pl.ds(wseq + g * CS, CS)],
                    ssem[i],
                )

    for i in range(NBUF):
        for b in range(B):
            pltpu.make_async_copy(
                rows_bufs[i].at[pl.ds(b * CS, CS)],
                out_hbm.at[b, pl.ds(0, CS)],
                ssem[i],
            ).wait()


def kernel(x, table, pos_encoding):
    return _embed(x.astype(jnp.int32), table, pos_encoding)


# speedup vs baseline: 1.1028x; 1.1028x over previous
import functools

import jax
import jax.numpy as jnp
from jax import lax
from jax.experimental import pallas as pl
from jax.experimental.pallas import tpu as pltpu
from jax.experimental.pallas import tpu_sc as plsc

B, S, D = 4, 4096, 768
LANES = 16
VPR = D // LANES
NW = 32
SEQ_PER_W = S // NW
CS = 8
NCHUNK = SEQ_PER_W // CS
NBUF = 4
AHEAD = 2

_mesh = plsc.VectorSubcoreMesh(core_axis_name="c", subcore_axis_name="s")

_scratch = (
    [pltpu.VMEM((NCHUNK * B * CS,), jnp.int32)]
    + [pltpu.VMEM((B * CS, D), jnp.float32) for _ in range(NBUF)]
    + [pltpu.VMEM((CS, D), jnp.float32) for _ in range(NBUF)]
    + [pltpu.SemaphoreType.DMA for _ in range(3 * NBUF)]
)


@functools.partial(
    pl.kernel,
    mesh=_mesh,
    out_type=jax.ShapeDtypeStruct((B, S, D), jnp.float32),
    scratch_types=_scratch,
)
def _embed(x_hbm, table_hbm, pos_hbm, out_hbm, idx_v, *bufs):
    rows_bufs = bufs[:NBUF]
    pos_bufs = bufs[NBUF:2 * NBUF]
    gsem = bufs[2 * NBUF:3 * NBUF]
    psem = bufs[3 * NBUF:4 * NBUF]
    ssem = bufs[4 * NBUF:5 * NBUF]

    cid = lax.axis_index("c")
    sid = lax.axis_index("s")
    wid = sid * 2 + cid
    wseq = wid * SEQ_PER_W

    for b in range(B):
        pltpu.async_copy(
            x_hbm.at[b, pl.ds(wseq, SEQ_PER_W)],
            idx_v.at[pl.ds(b * SEQ_PER_W, SEQ_PER_W)],
            gsem[0],
        )
    for b in range(B):
        pltpu.make_async_copy(
            x_hbm.at[0, pl.ds(0, SEQ_PER_W)],
            idx_v.at[pl.ds(0, SEQ_PER_W)],
            gsem[0],
        ).wait()

    def fire(g, j):
        pltpu.async_copy(
            pos_hbm.at[pl.ds(wseq + g * CS, CS)], pos_bufs[j], psem[j]
        )
        for b in range(B):
            pltpu.async_copy(
                table_hbm.at[idx_v.at[pl.ds(b * SEQ_PER_W + g * CS, CS)]],
                rows_bufs[j].at[pl.ds(b * CS, CS)],
                gsem[j],
            )

    for g in range(AHEAD):
        fire(g, g % NBUF)

    @pl.loop(0, NCHUNK, step=NBUF)
    def _main(gout):
        for i in range(NBUF):
            g = gout + i
            for b in range(B):
                pltpu.make_async_copy(
                    table_hbm.at[pl.ds(0, CS)],
                    rows_bufs[i].at[pl.ds(b * CS, CS)],
                    gsem[i],
                ).wait()
            pltpu.make_async_copy(
                pos_hbm.at[pl.ds(0, CS)], pos_bufs[i], psem[i]
            ).wait()

            j = (i + AHEAD) % NBUF

            @pl.when(g + AHEAD < NCHUNK)
            def _fire_ahead(g=g, j=j):
                @pl.when(g + AHEAD >= NBUF)
                def _drain_store():
                    for b in range(B):
                        pltpu.make_async_copy(
                            rows_bufs[j].at[pl.ds(b * CS, CS)],
                            out_hbm.at[b, pl.ds(0, CS)],
                            ssem[j],
                        ).wait()

                fire(g + AHEAD, j)

            rows = rows_bufs[i]
            pos = pos_bufs[i]

            def add_row(jj, c2, rows=rows, pos=pos):
                def add_vec(k, c3):
                    off = k * LANES
                    p = pos[jj, pl.ds(off, LANES)]
                    for b in range(B):
                        plsc.addupdate(rows.at[b * CS + jj, pl.ds(off, LANES)], p)
                    return c3

                return lax.fori_loop(0, VPR, add_vec, c2)

            lax.fori_loop(0, CS, add_row, 0)

            for b in range(B):
                pltpu.async_copy(
                    rows.at[pl.ds(b * CS, CS)],
                    out_hbm.at[b, pl.ds(wseq + g * CS, CS)],
                    ssem[i],
                )

    for i in range(NBUF):
        for b in range(B):
            pltpu.make_async_copy(
                rows_bufs[i].at[pl.ds(b * CS, CS)],
                out_hbm.at[b, pl.ds(0, CS)],
                ssem[i],
            ).wait()


def kernel(x, table, pos_encoding):
    return _embed(x.astype(jnp.int32), table, pos_encoding)


# speedup vs baseline: 1.1315x; 1.0260x over previous
import functools

import jax
import jax.numpy as jnp
from jax import lax
from jax.experimental import pallas as pl
from jax.experimental.pallas import tpu as pltpu
from jax.experimental.pallas import tpu_sc as plsc

B, S, D = 4, 4096, 768
LANES = 16
VPR = D // LANES
NW = 32
SEQ_PER_W = S // NW
CS = 8
NCHUNK = SEQ_PER_W // CS
NBUF = 4
AHEAD = 2

_mesh = plsc.VectorSubcoreMesh(core_axis_name="c", subcore_axis_name="s")

_scratch = (
    [pltpu.VMEM((NCHUNK * B * CS,), jnp.int32)]
    + [pltpu.VMEM((B * CS, D), jnp.float32) for _ in range(NBUF)]
    + [pltpu.VMEM((CS, D), jnp.float32) for _ in range(NBUF)]
    + [pltpu.SemaphoreType.DMA for _ in range(3 * NBUF)]
)


@functools.partial(
    pl.kernel,
    mesh=_mesh,
    out_type=jax.ShapeDtypeStruct((B, S, D), jnp.float32),
    scratch_types=_scratch,
)
def _embed(x_hbm, table_hbm, pos_hbm, out_hbm, idx_v, *bufs):
    rows_bufs = bufs[:NBUF]
    pos_bufs = bufs[NBUF:2 * NBUF]
    gsem = bufs[2 * NBUF:3 * NBUF]
    psem = bufs[3 * NBUF:4 * NBUF]
    ssem = bufs[4 * NBUF:5 * NBUF]

    cid = lax.axis_index("c")
    sid = lax.axis_index("s")
    wid = sid * 2 + cid
    wseq = wid * SEQ_PER_W

    for b in range(B):
        pltpu.async_copy(
            x_hbm.at[b, pl.ds(wseq, SEQ_PER_W)],
            idx_v.at[pl.ds(b * SEQ_PER_W, SEQ_PER_W)],
            gsem[0],
        )
    for b in range(B):
        pltpu.make_async_copy(
            x_hbm.at[0, pl.ds(0, SEQ_PER_W)],
            idx_v.at[pl.ds(0, SEQ_PER_W)],
            gsem[0],
        ).wait()

    def fire(g, j):
        pltpu.async_copy(
            pos_hbm.at[pl.ds(wseq + g * CS, CS)], pos_bufs[j], psem[j]
        )
        for b in range(B):
            pltpu.async_copy(
                table_hbm.at[idx_v.at[pl.ds(b * SEQ_PER_W + g * CS, CS)]],
                rows_bufs[j].at[pl.ds(b * CS, CS)],
                gsem[j],
            )

    for g in range(AHEAD):
        fire(g, g % NBUF)

    @pl.loop(0, NCHUNK, step=NBUF)
    def _main(gout):
        for i in range(NBUF):
            g = gout + i
            for b in range(B):
                pltpu.make_async_copy(
                    table_hbm.at[pl.ds(0, CS)],
                    rows_bufs[i].at[pl.ds(b * CS, CS)],
                    gsem[i],
                ).wait()
            pltpu.make_async_copy(
                pos_hbm.at[pl.ds(0, CS)], pos_bufs[i], psem[i]
            ).wait()

            j = (i + AHEAD) % NBUF

            @pl.when(g + AHEAD < NCHUNK)
            def _fire_ahead(g=g, j=j):
                @pl.when(g + AHEAD >= NBUF)
                def _drain_store():
                    for b in range(B):
                        pltpu.make_async_copy(
                            rows_bufs[j].at[pl.ds(b * CS, CS)],
                            out_hbm.at[b, pl.ds(0, CS)],
                            ssem[j],
                        ).wait()

                fire(g + AHEAD, j)

            rows = rows_bufs[i]
            pos = pos_bufs[i]

            def add_row(jj, c2, rows=rows, pos=pos):
                @plsc.parallel_loop(0, VPR, 1, unroll=4)
                def add_vec(k):
                    off = k * LANES
                    p = pos[jj, pl.ds(off, LANES)]
                    for b in range(B):
                        plsc.addupdate(rows.at[b * CS + jj, pl.ds(off, LANES)], p)

                return c2

            lax.fori_loop(0, CS, add_row, 0)

            for b in range(B):
                pltpu.async_copy(
                    rows.at[pl.ds(b * CS, CS)],
                    out_hbm.at[b, pl.ds(wseq + g * CS, CS)],
                    ssem[i],
                )

    for i in range(NBUF):
        for b in range(B):
            pltpu.make_async_copy(
                rows_bufs[i].at[pl.ds(b * CS, CS)],
                out_hbm.at[b, pl.ds(0, CS)],
                ssem[i],
            ).wait()


def kernel(x, table, pos_encoding):
    return _embed(x.astype(jnp.int32), table, pos_encoding)
